# baseline (speedup 1.0000x reference)
import functools

import jax
import jax.numpy as jnp
from jax import lax
from jax.experimental import pallas as pl
from jax.experimental.pallas import tpu as pltpu
from jax.experimental.pallas import tpu_sc as plsc

N_NODES = 10000
N_EDGES = 5000
NNZ = 320000
D = 128
DH = 64

NC = 2
NS = 16
CHUNK = 128
NNZ_TILE = 20480
NCHUNK = NNZ_TILE // CHUNK
HCHUNK = NCHUNK // 2
NNZ_PAD = NNZ_TILE * NS

E_TILE = 320
E_PAD = E_TILE * NS
V_TILE = 640
V_PAD = V_TILE * NS

MM_BLK = 1000


def _matmul_body(x_ref, w_ref, o0_ref, o1_ref):
    xw = lax.dot_general(x_ref[...], w_ref[...],
                         (((1,), (1,)), ((), ())),
                         preferred_element_type=jnp.float32)
    o0_ref[...] = xw[:, :DH]
    o1_ref[...] = xw[:, DH:]


def _tc_matmul(x, w):
    grid = N_NODES // MM_BLK
    return pl.pallas_call(
        _matmul_body,
        grid=(grid,),
        in_specs=[
            pl.BlockSpec((MM_BLK, D), lambda i: (i, 0)),
            pl.BlockSpec((D, D), lambda i: (0, 0)),
        ],
        out_specs=[
            pl.BlockSpec((MM_BLK, DH), lambda i: (i, 0)),
            pl.BlockSpec((MM_BLK, DH), lambda i: (i, 0)),
        ],
        out_shape=[
            jax.ShapeDtypeStruct((N_NODES, DH), jnp.float32),
            jax.ShapeDtypeStruct((N_NODES, DH), jnp.float32),
        ],
    )(x, w)


def _sc_body(xw0, xw1, nidx, eidx, se, sv,
             out0, out1,
             xe_acc, xv_acc,
             nidx_v, eidx_v,
             gb0, gb1, gb2, gb3, vbuf, sbuf,
             gs0, gs1, gs2, gs3, ss0, ss1, ss2, ss3):
    c = lax.axis_index("c")
    s = lax.axis_index("s")
    gbufs = (gb0, gb1, gb2, gb3)
    gsems = (gs0, gs1, gs2, gs3)
    ssems = (ss0, ss1, ss2, ss3)

    def zbody(r, _):
        z = jnp.zeros((16,), jnp.float32)
        for g in range(DH // 16):
            vbuf[r, pl.ds(g * 16, 16)] = z
        return 0
    lax.fori_loop(0, CHUNK, zbody, 0)
    for piece0, rows in ((0, CHUNK), (CHUNK, CHUNK), (2 * CHUNK, E_TILE - 2 * CHUNK)):
        pltpu.sync_copy(vbuf.at[pl.ds(0, rows)],
                        xe_acc.at[pl.ds(s * E_TILE + piece0, rows)])
    for p in range(V_TILE // CHUNK):
        pltpu.sync_copy(vbuf, xv_acc.at[pl.ds(s * V_TILE + p * CHUNK, CHUNK)])

    def _scale_rows(acc, scale_hbm, row0, nrows):
        pltpu.sync_copy(acc.at[pl.ds(row0, nrows)], vbuf.at[pl.ds(0, nrows)])
        pltpu.sync_copy(scale_hbm.at[pl.ds(row0, nrows)],
                        sbuf.at[pl.ds(0, nrows)])

        def body(r16, _):
            base = r16 * 16
            svec = sbuf[pl.ds(base, 16)]
            for k in range(16):
                sk = lax.gather(
                    svec, jnp.full((16, 1), k, jnp.int32),
                    lax.GatherDimensionNumbers(offset_dims=(),
                                               collapsed_slice_dims=(0,),
                                               start_index_map=(0,)),
                    slice_sizes=(1,),
                    mode=lax.GatherScatterMode.PROMISE_IN_BOUNDS)
                r = base + k
                for g in range(DH // 16):
                    vbuf[r, pl.ds(g * 16, 16)] = vbuf[r, pl.ds(g * 16, 16)] * sk
            return 0

        lax.fori_loop(0, nrows // 16, body, 0)

    def _gather_scatter_phase(src0, src1, gidx_v, dst_acc, sidx_v, two_src):
        def start_gather(j, b, guard):
            if two_src:
                @pl.when(jnp.logical_and(c == 0, guard))
                def _():
                    pltpu.async_copy(src0.at[gidx_v.at[j]], gbufs[b], gsems[b])

                @pl.when(jnp.logical_and(c == 1, guard))
                def _():
                    pltpu.async_copy(src1.at[gidx_v.at[j]], gbufs[b], gsems[b])
            else:
                @pl.when(guard)
                def _():
                    pltpu.async_copy(src0.at[gidx_v.at[j]], gbufs[b], gsems[b])

        def wait_gather(b):
            pltpu.make_async_copy(src0.at[pl.ds(0, CHUNK)], gbufs[b],
                                  gsems[b]).wait()

        def wait_scatter(j, b):
            pltpu.make_async_copy(gbufs[b], dst_acc.at[sidx_v.at[j]],
                                  ssems[b]).wait()

        start_gather(jnp.int32(0), 0, jnp.bool_(True))
        start_gather(jnp.int32(1), 1, jnp.bool_(True))

        def quad(j4, _):
            for b in range(4):
                j = j4 * 4 + b
                wait_gather(b)
                pltpu.async_copy(gbufs[b], dst_acc.at[sidx_v.at[j]], ssems[b],
                                 add=True)
                jn = j + 2
                bn = (b + 2) % 4
                @pl.when(jnp.logical_and(jn < HCHUNK, j >= 2))
                def _():
                    wait_scatter(j, bn)
                start_gather(jn, bn, jn < HCHUNK)
            return 0
        lax.fori_loop(0, HCHUNK // 4, quad, 0)
        for b in range(4):
            wait_scatter(jnp.int32(0), b)

    plsc.subcore_barrier()
    for h in range(2):
        pltpu.sync_copy(nidx.at[s, pl.ds(h * HCHUNK, HCHUNK)], nidx_v)
        pltpu.sync_copy(eidx.at[s, pl.ds(h * HCHUNK, HCHUNK)], eidx_v)
        _gather_scatter_phase(xw0, xw1, nidx_v, xe_acc, eidx_v, True)
    plsc.subcore_barrier()

    for piece0, rows in ((0, CHUNK), (CHUNK, CHUNK), (2 * CHUNK, E_TILE - 2 * CHUNK)):
        row0 = s * E_TILE + piece0
        _scale_rows(xe_acc, se, row0, rows)
        pltpu.sync_copy(vbuf.at[pl.ds(0, rows)], xe_acc.at[pl.ds(row0, rows)])
    plsc.subcore_barrier()

    for h in range(2):
        pltpu.sync_copy(nidx.at[s, pl.ds(h * HCHUNK, HCHUNK)], nidx_v)
        pltpu.sync_copy(eidx.at[s, pl.ds(h * HCHUNK, HCHUNK)], eidx_v)
        _gather_scatter_phase(xe_acc, xe_acc, eidx_v, xv_acc, nidx_v, False)
    plsc.subcore_barrier()

    for p in range(V_TILE // CHUNK):
        row0 = s * V_TILE + p * CHUNK
        _scale_rows(xv_acc, sv, row0, CHUNK)

        @pl.when(c == 0)
        def _():
            pltpu.sync_copy(vbuf, out0.at[pl.ds(row0, CHUNK)])

        @pl.when(c == 1)
        def _():
            pltpu.sync_copy(vbuf, out1.at[pl.ds(row0, CHUNK)])


@functools.partial(
    pl.kernel,
    out_type=[
        jax.ShapeDtypeStruct((V_PAD, DH), jnp.float32),
        jax.ShapeDtypeStruct((V_PAD, DH), jnp.float32),
    ],
    mesh=plsc.VectorSubcoreMesh(core_axis_name="c", subcore_axis_name="s"),
    scratch_types=[
        pltpu.VMEM_SHARED((E_PAD, DH), jnp.float32),
        pltpu.VMEM_SHARED((V_PAD, DH), jnp.float32),
        pltpu.VMEM((HCHUNK, CHUNK), jnp.int32),
        pltpu.VMEM((HCHUNK, CHUNK), jnp.int32),
        pltpu.VMEM((CHUNK, DH), jnp.float32),
        pltpu.VMEM((CHUNK, DH), jnp.float32),
        pltpu.VMEM((CHUNK, DH), jnp.float32),
        pltpu.VMEM((CHUNK, DH), jnp.float32),
        pltpu.VMEM((CHUNK, DH), jnp.float32),
        pltpu.VMEM((CHUNK,), jnp.float32),
        pltpu.SemaphoreType.DMA,
        pltpu.SemaphoreType.DMA,
        pltpu.SemaphoreType.DMA,
        pltpu.SemaphoreType.DMA,
        pltpu.SemaphoreType.DMA,
        pltpu.SemaphoreType.DMA,
        pltpu.SemaphoreType.DMA,
        pltpu.SemaphoreType.DMA,
    ],
    compiler_params=pltpu.CompilerParams(use_tc_tiling_on_sc=False),
)
def _sc_aggregate(xw0, xw1, nidx, eidx, se, sv, out0, out1, *rest):
    _sc_body(xw0, xw1, nidx, eidx, se, sv, out0, out1, *rest)


def kernel(X, W, node_idx, edge_idx, degE, degV, Wdiag):
    nidx = jnp.zeros((NNZ_PAD,), jnp.int32).at[:NNZ].set(node_idx)
    eidx = jnp.full((NNZ_PAD,), N_EDGES, jnp.int32).at[:NNZ].set(edge_idx)
    nidx = nidx.reshape(NS, NCHUNK, CHUNK)
    eidx = eidx.reshape(NS, NCHUNK, CHUNK)
    se = jnp.zeros((E_PAD,), jnp.float32).at[:N_EDGES].set(degE * Wdiag)
    sv = jnp.zeros((V_PAD,), jnp.float32).at[:N_NODES].set(degV)

    xw0, xw1 = _tc_matmul(X, W)
    o0, o1 = _sc_aggregate(xw0, xw1, nidx, eidx, se, sv)
    return jnp.concatenate([o0[:N_NODES], o1[:N_NODES]], axis=1)


# speedup vs baseline: 1.6781x; 1.6781x over previous
import functools

import jax
import jax.numpy as jnp
from jax import lax
from jax.experimental import pallas as pl
from jax.experimental.pallas import tpu as pltpu
from jax.experimental.pallas import tpu_sc as plsc

N_NODES = 10000
N_EDGES = 5000
NNZ = 320000
D = 128
DH = 64

NC = 2
NS = 16
CHUNK = 128
NNZ_TILE = 20480
NCHUNK = NNZ_TILE // CHUNK
HCHUNK = NCHUNK // 2
NNZ_PAD = NNZ_TILE * NS

E_TILE = 320
E_PAD = E_TILE * NS
V_TILE = 640
V_PAD = V_TILE * NS

MM_BLK = 1000


def _matmul_body(x_ref, w_ref, o0_ref, o1_ref):
    xw = lax.dot_general(x_ref[...], w_ref[...],
                         (((1,), (1,)), ((), ())),
                         preferred_element_type=jnp.float32)
    o0_ref[...] = xw[:, :DH]
    o1_ref[...] = xw[:, DH:]


def _tc_matmul(x, w):
    grid = N_NODES // MM_BLK
    return pl.pallas_call(
        _matmul_body,
        grid=(grid,),
        in_specs=[
            pl.BlockSpec((MM_BLK, D), lambda i: (i, 0)),
            pl.BlockSpec((D, D), lambda i: (0, 0)),
        ],
        out_specs=[
            pl.BlockSpec((MM_BLK, DH), lambda i: (i, 0)),
            pl.BlockSpec((MM_BLK, DH), lambda i: (i, 0)),
        ],
        out_shape=[
            jax.ShapeDtypeStruct((N_NODES, DH), jnp.float32),
            jax.ShapeDtypeStruct((N_NODES, DH), jnp.float32),
        ],
    )(x, w)


def _sc_body(xw0, xw1, nidx, eidx, se, sv,
             out0, out1,
             xe_acc, xv_acc,
             nidx_v, eidx_v,
             gb0, gb1, gb2, gb3, vbuf, sbuf,
             gs0, gs1, gs2, gs3, ss0, ss1, ss2, ss3):
    c = lax.axis_index("c")
    s = lax.axis_index("s")
    gbufs = (gb0, gb1, gb2, gb3)
    gsems = (gs0, gs1, gs2, gs3)
    ssems = (ss0, ss1, ss2, ss3)

    def zbody(r, _):
        z = jnp.zeros((16,), jnp.float32)
        for g in range(DH // 16):
            vbuf[r, pl.ds(g * 16, 16)] = z
        return 0
    lax.fori_loop(0, CHUNK, zbody, 0)
    for piece0, rows in ((0, CHUNK), (CHUNK, CHUNK), (2 * CHUNK, E_TILE - 2 * CHUNK)):
        pltpu.sync_copy(vbuf.at[pl.ds(0, rows)],
                        xe_acc.at[pl.ds(s * E_TILE + piece0, rows)])
    for p in range(V_TILE // CHUNK):
        pltpu.sync_copy(vbuf, xv_acc.at[pl.ds(s * V_TILE + p * CHUNK, CHUNK)])

    def _scale_rows(acc, scale_hbm, row0, nrows):
        pltpu.sync_copy(acc.at[pl.ds(row0, nrows)], vbuf.at[pl.ds(0, nrows)])
        pltpu.sync_copy(scale_hbm.at[pl.ds(row0, nrows)],
                        sbuf.at[pl.ds(0, nrows)])

        def body(r16, _):
            base = r16 * 16
            svec = sbuf[pl.ds(base, 16)]
            for k in range(16):
                sk = lax.gather(
                    svec, jnp.full((16, 1), k, jnp.int32),
                    lax.GatherDimensionNumbers(offset_dims=(),
                                               collapsed_slice_dims=(0,),
                                               start_index_map=(0,)),
                    slice_sizes=(1,),
                    mode=lax.GatherScatterMode.PROMISE_IN_BOUNDS)
                r = base + k
                for g in range(DH // 16):
                    vbuf[r, pl.ds(g * 16, 16)] = vbuf[r, pl.ds(g * 16, 16)] * sk
            return 0

        lax.fori_loop(0, nrows // 16, body, 0)

    def _gather_scatter_phase(src0, src1, gidx_v, dst_acc, sidx_v, two_src):
        def start_gather(j, b, guard):
            if two_src:
                @pl.when(jnp.logical_and(c == 0, guard))
                def _():
                    pltpu.async_copy(src0.at[gidx_v.at[j]], gbufs[b], gsems[b])

                @pl.when(jnp.logical_and(c == 1, guard))
                def _():
                    pltpu.async_copy(src1.at[gidx_v.at[j]], gbufs[b], gsems[b])
            else:
                @pl.when(guard)
                def _():
                    pltpu.async_copy(src0.at[gidx_v.at[j]], gbufs[b], gsems[b])

        def wait_gather(b):
            pltpu.make_async_copy(src0.at[pl.ds(0, CHUNK)], gbufs[b],
                                  gsems[b]).wait()

        def wait_scatter(j, b):
            pltpu.make_async_copy(gbufs[b], dst_acc.at[sidx_v.at[j]],
                                  ssems[b]).wait()

        start_gather(jnp.int32(0), 0, jnp.bool_(True))
        start_gather(jnp.int32(1), 1, jnp.bool_(True))

        def quad(j4, _):
            for b in range(4):
                j = j4 * 4 + b
                wait_gather(b)
                pltpu.async_copy(gbufs[b], dst_acc.at[sidx_v.at[j]], ssems[b],
                                 add=True)
                jn = j + 2
                bn = (b + 2) % 4
                @pl.when(jnp.logical_and(jn < HCHUNK, j >= 2))
                def _():
                    wait_scatter(j, bn)
                start_gather(jn, bn, jn < HCHUNK)
            return 0
        lax.fori_loop(0, HCHUNK // 4, quad, 0)
        for b in range(4):
            wait_scatter(jnp.int32(0), b)

    plsc.subcore_barrier()
    for h in range(2):
        pltpu.sync_copy(nidx.at[s, pl.ds(h * HCHUNK, HCHUNK)], nidx_v)
        pltpu.sync_copy(eidx.at[s, pl.ds(h * HCHUNK, HCHUNK)], eidx_v)
        _gather_scatter_phase(xw0, xw1, nidx_v, xe_acc, eidx_v, True)
    plsc.subcore_barrier()

    for piece0, rows in ((0, CHUNK), (CHUNK, CHUNK), (2 * CHUNK, E_TILE - 2 * CHUNK)):
        row0 = s * E_TILE + piece0
        _scale_rows(xe_acc, se, row0, rows)
        pltpu.sync_copy(vbuf.at[pl.ds(0, rows)], xe_acc.at[pl.ds(row0, rows)])
    plsc.subcore_barrier()

    for h in range(2):
        pltpu.sync_copy(nidx.at[s, pl.ds(h * HCHUNK, HCHUNK)], nidx_v)
        pltpu.sync_copy(eidx.at[s, pl.ds(h * HCHUNK, HCHUNK)], eidx_v)
        _gather_scatter_phase(xe_acc, xe_acc, eidx_v, xv_acc, nidx_v, False)
    plsc.subcore_barrier()

    for p in range(V_TILE // CHUNK):
        row0 = s * V_TILE + p * CHUNK
        _scale_rows(xv_acc, sv, row0, CHUNK)

        @pl.when(c == 0)
        def _():
            pltpu.sync_copy(vbuf, out0.at[pl.ds(row0, CHUNK)])

        @pl.when(c == 1)
        def _():
            pltpu.sync_copy(vbuf, out1.at[pl.ds(row0, CHUNK)])


@functools.partial(
    pl.kernel,
    out_type=[
        jax.ShapeDtypeStruct((V_PAD, DH), jnp.float32),
        jax.ShapeDtypeStruct((V_PAD, DH), jnp.float32),
    ],
    mesh=plsc.VectorSubcoreMesh(core_axis_name="c", subcore_axis_name="s"),
    scratch_types=[
        pltpu.VMEM_SHARED((E_PAD, DH), jnp.float32),
        pltpu.VMEM_SHARED((V_PAD, DH), jnp.float32),
        pltpu.VMEM((HCHUNK, CHUNK), jnp.int32),
        pltpu.VMEM((HCHUNK, CHUNK), jnp.int32),
        pltpu.VMEM((CHUNK, DH), jnp.float32),
        pltpu.VMEM((CHUNK, DH), jnp.float32),
        pltpu.VMEM((CHUNK, DH), jnp.float32),
        pltpu.VMEM((CHUNK, DH), jnp.float32),
        pltpu.VMEM((CHUNK, DH), jnp.float32),
        pltpu.VMEM((CHUNK,), jnp.float32),
        pltpu.SemaphoreType.DMA,
        pltpu.SemaphoreType.DMA,
        pltpu.SemaphoreType.DMA,
        pltpu.SemaphoreType.DMA,
        pltpu.SemaphoreType.DMA,
        pltpu.SemaphoreType.DMA,
        pltpu.SemaphoreType.DMA,
        pltpu.SemaphoreType.DMA,
    ],
    compiler_params=pltpu.CompilerParams(use_tc_tiling_on_sc=False),
)
def _sc_aggregate(xw0, xw1, nidx, eidx, se, sv, out0, out1, *rest):
    _sc_body(xw0, xw1, nidx, eidx, se, sv, out0, out1, *rest)


def kernel(X, W, node_idx, edge_idx, degE, degV, Wdiag):
    npadv = jnp.arange(NNZ_PAD - NNZ, dtype=jnp.int32) % N_NODES
    epadv = N_EDGES + jnp.arange(NNZ_PAD - NNZ, dtype=jnp.int32) % (E_PAD - N_EDGES)
    nidx = jnp.concatenate([node_idx, npadv])
    eidx = jnp.concatenate([edge_idx, epadv])
    nidx = nidx.reshape(NS, NCHUNK, CHUNK)
    eidx = eidx.reshape(NS, NCHUNK, CHUNK)
    se = jnp.zeros((E_PAD,), jnp.float32).at[:N_EDGES].set(degE * Wdiag)
    sv = jnp.zeros((V_PAD,), jnp.float32).at[:N_NODES].set(degV)

    xw0, xw1 = _tc_matmul(X, W)
    o0, o1 = _sc_aggregate(xw0, xw1, nidx, eidx, se, sv)
    return jnp.concatenate([o0[:N_NODES], o1[:N_NODES]], axis=1)
